# baseline (speedup 1.0000x reference)
import functools

import jax
import jax.numpy as jnp
import numpy as np
from jax import lax
from jax.experimental import pallas as pl
from jax.experimental.pallas import tpu as pltpu
from jax.experimental.pallas import tpu_sc as plsc

N = 10000
DEG = 32
D = 128
UNITS = 128
OUT = 3

NC = 2
NS = 16
NW = NC * NS
RPW = 320
NPAD = NW * RPW
BATCH = 4
UW = UNITS // 2

_SELU_SCALE = 1.0507009873554805
_SELU_ALPHA = 1.6732632423543772

_PERM = np.concatenate([
    np.concatenate([c * 32 + np.arange(0, 32, 2), c * 32 + np.arange(1, 32, 2)])
    for c in range(UNITS // 32)
]).astype(np.int32)


def _prep_w(w, d_in):
    return w.reshape(DEG, d_in, UNITS).transpose(1, 0, 2).reshape(
        d_in, DEG * UNITS)


def _matmul_packed(x, w):
    x = x.astype(jnp.bfloat16)
    w = w.astype(jnp.bfloat16)
    m = x.shape[0]

    def mm(x_ref, w_ref, o_ref):
        o_ref[...] = jnp.dot(x_ref[...], w_ref[...],
                             preferred_element_type=jnp.float32)

    return pl.pallas_call(
        mm,
        grid=(DEG,),
        in_specs=[
            pl.BlockSpec((m, D), lambda kk: (0, 0)),
            pl.BlockSpec((D, UNITS), lambda kk: (0, kk)),
        ],
        out_specs=pl.BlockSpec((m, UNITS), lambda kk: (kk, 0)),
        out_shape=jax.ShapeDtypeStruct((DEG * m, UNITS), jnp.float32),
    )(x, w)


def _head(h, w3, b3, bt):
    m = h.shape[0]

    def hk(h_ref, w_ref, b_ref, o_ref):
        logits = jnp.dot(h_ref[...], w_ref[...],
                         preferred_element_type=jnp.float32) + b_ref[...]
        mx = jnp.max(logits, axis=-1, keepdims=True)
        e = jnp.exp(logits - mx)
        o_ref[...] = e / jnp.sum(e, axis=-1, keepdims=True)

    return pl.pallas_call(
        hk,
        grid=(m // bt,),
        in_specs=[
            pl.BlockSpec((bt, UNITS), lambda i: (i, 0)),
            pl.BlockSpec((UNITS, OUT), lambda i: (0, 0)),
            pl.BlockSpec((1, OUT), lambda i: (0, 0)),
        ],
        out_specs=pl.BlockSpec((bt, OUT), lambda i: (i, 0)),
        out_shape=jax.ShapeDtypeStruct((m, OUT), jnp.float32),
    )(h, w3, b3)


RING = 2
GRP = 80
NG = RPW // GRP


def _make_sc_layer(stride):
    mesh = plsc.VectorSubcoreMesh(core_axis_name="c", subcore_axis_name="s")

    @functools.partial(
        pl.kernel,
        mesh=mesh,
        out_type=jax.ShapeDtypeStruct((NPAD, UNITS), jnp.float32),
        scratch_types=[
            pltpu.VMEM((DEG, RPW), jnp.int32),
            pltpu.VMEM((NG, DEG, GRP), jnp.int32),
            pltpu.VMEM((RING, GRP, UNITS), jnp.float32),
            pltpu.VMEM((RING, GRP, UNITS), jnp.float32),
            pltpu.VMEM((UNITS,), jnp.float32),
            pltpu.SemaphoreType.DMA,
            pltpu.SemaphoreType.DMA,
            pltpu.SemaphoreType.DMA,
            pltpu.SemaphoreType.DMA,
        ],
    )
    def sc_layer(z_hbm, gt_hbm, b_hbm, out_hbm, gt_v, idx_all, acc_v, o_v,
                 b_v, sem_g0, sem_g1, sem_o0, sem_o1):
        sem_g = (sem_g0, sem_g1)
        sem_o = (sem_o0, sem_o1)
        wid = lax.axis_index("s") * NC + lax.axis_index("c")
        base = wid * RPW
        pltpu.sync_copy(b_hbm, b_v)
        pltpu.sync_copy(gt_hbm.at[wid], gt_v)

        def idx_body(g, carry):
            for k in range(DEG):
                for h in range(GRP // 16):
                    idx_all[g, k, pl.ds(h * 16, 16)] = (
                        gt_v[k, pl.ds(g * GRP + h * 16, 16)] + k * stride)
            return carry

        lax.fori_loop(0, NG, idx_body, 0)

        def init_bias(slot):
            def bias_body(rb, carry):
                for rr in range(4):
                    for c in range(UNITS // 16):
                        acc_v[slot, rb * 4 + rr, pl.ds(c * 16, 16)] = (
                            b_v[pl.ds(c * 16, 16)])
                return carry
            lax.fori_loop(0, GRP // 4, bias_body, 0)

        def fire_group(g, slot):
            for k in range(DEG):
                pltpu.async_copy(z_hbm.at[idx_all.at[g, k]], acc_v.at[slot],
                                 sem_g[slot], add=True)

        def wait_group(slot):
            for k in range(DEG):
                pltpu.make_async_copy(
                    z_hbm.at[idx_all.at[0, 0]], acc_v.at[slot],
                    sem_g[slot]).wait()

        def selu(acc):
            return jnp.where(
                acc > 0.0,
                _SELU_SCALE * acc,
                (_SELU_SCALE * _SELU_ALPHA) * (jnp.exp(acc) - 1.0))

        for slot in range(RING):
            init_bias(slot)
        for g in range(RING):
            fire_group(g, g)

        for g in range(NG):
            slot = g % RING
            wait_group(slot)
            if g >= RING:
                pltpu.make_async_copy(
                    o_v.at[slot], out_hbm.at[pl.ds(base, GRP)],
                    sem_o[slot]).wait()

            def ep_body(rb, carry, slot=slot):
                for rr in range(4):
                    for c in range(UNITS // 16):
                        o_v[slot, rb * 4 + rr, pl.ds(c * 16, 16)] = selu(
                            acc_v[slot, rb * 4 + rr, pl.ds(c * 16, 16)])
                return carry

            lax.fori_loop(0, GRP // 4, ep_body, 0)
            if g + RING < NG:
                init_bias(slot)
                fire_group(g + RING, slot)
            pltpu.async_copy(
                o_v.at[slot], out_hbm.at[pl.ds(base + g * GRP, GRP)],
                sem_o[slot])

        for g in range(NG - RING, NG):
            slot = g % RING
            pltpu.make_async_copy(
                o_v.at[slot], out_hbm.at[pl.ds(base, GRP)],
                sem_o[slot]).wait()

    return sc_layer


_sc_layer_1 = _make_sc_layer(NPAD)
_sc_layer_2 = _make_sc_layer(NPAD)


def kernel(X, G, W1, b1, W2, b2, W3, b3):
    Xp = jnp.zeros((NPAD, D), jnp.float32).at[:N].set(X)
    GpT = jnp.zeros((DEG, NPAD), jnp.int32).at[:, :N].set(G.T)
    GpT = GpT.reshape(DEG, NW, RPW).transpose(1, 0, 2)
    W1P = _prep_w(W1, D)
    W2P = _prep_w(W2, UNITS)

    Z1 = _matmul_packed(Xp, W1P)
    H1 = _sc_layer_1(Z1, GpT, b1)
    Z2 = _matmul_packed(H1, W2P)
    H2 = _sc_layer_2(Z2, GpT, b2)
    P = _head(H2, W3, b3.reshape(1, OUT), 1024)
    return P[:N - 1]


# speedup vs baseline: 1.1602x; 1.1602x over previous
import functools

import jax
import jax.numpy as jnp
import numpy as np
from jax import lax
from jax.experimental import pallas as pl
from jax.experimental.pallas import tpu as pltpu
from jax.experimental.pallas import tpu_sc as plsc

N = 10000
DEG = 32
D = 128
UNITS = 128
OUT = 3

NC = 2
NS = 16
NW = NC * NS
RPW = 320
NPAD = NW * RPW
BATCH = 4
UW = UNITS // 2

_SELU_SCALE = 1.0507009873554805
_SELU_ALPHA = 1.6732632423543772

_PERM = np.concatenate([
    np.concatenate([c * 32 + np.arange(0, 32, 2), c * 32 + np.arange(1, 32, 2)])
    for c in range(UNITS // 32)
]).astype(np.int32)


def _prep_w(w, d_in):
    return w.reshape(DEG, d_in, UNITS).transpose(1, 0, 2).reshape(
        d_in, DEG * UNITS)


def _matmul_packed(x, w):
    x = x.astype(jnp.bfloat16)
    w = w.astype(jnp.bfloat16)
    m = x.shape[0]

    def mm(x_ref, w_ref, o_ref):
        o_ref[...] = jnp.dot(x_ref[...], w_ref[...],
                             preferred_element_type=jnp.float32)

    return pl.pallas_call(
        mm,
        grid=(DEG,),
        in_specs=[
            pl.BlockSpec((m, D), lambda kk: (0, 0)),
            pl.BlockSpec((D, UNITS), lambda kk: (0, kk)),
        ],
        out_specs=pl.BlockSpec((m, UNITS), lambda kk: (kk, 0)),
        out_shape=jax.ShapeDtypeStruct((DEG * m, UNITS), jnp.float32),
    )(x, w)


def _head(h, w3, b3, bt):
    m = h.shape[0]

    def hk(h_ref, w_ref, b_ref, o_ref):
        logits = jnp.dot(h_ref[...], w_ref[...],
                         preferred_element_type=jnp.float32) + b_ref[...]
        mx = jnp.max(logits, axis=-1, keepdims=True)
        e = jnp.exp(logits - mx)
        o_ref[...] = e / jnp.sum(e, axis=-1, keepdims=True)

    return pl.pallas_call(
        hk,
        grid=(m // bt,),
        in_specs=[
            pl.BlockSpec((bt, UNITS), lambda i: (i, 0)),
            pl.BlockSpec((UNITS, OUT), lambda i: (0, 0)),
            pl.BlockSpec((1, OUT), lambda i: (0, 0)),
        ],
        out_specs=pl.BlockSpec((bt, OUT), lambda i: (i, 0)),
        out_shape=jax.ShapeDtypeStruct((m, OUT), jnp.float32),
    )(h, w3, b3)


RING = 2
GRP = 32
NG = RPW // GRP


def _make_sc_layer(stride):
    mesh = plsc.VectorSubcoreMesh(core_axis_name="c", subcore_axis_name="s")

    @functools.partial(
        pl.kernel,
        mesh=mesh,
        out_type=jax.ShapeDtypeStruct((NPAD, UNITS), jnp.float32),
        scratch_types=[
            pltpu.VMEM((DEG, RPW), jnp.int32),
            pltpu.VMEM((NG, DEG, GRP), jnp.int32),
            pltpu.VMEM((RING, GRP, UNITS), jnp.float32),
            pltpu.VMEM((RING, GRP, UNITS), jnp.float32),
            pltpu.VMEM((UNITS,), jnp.float32),
            pltpu.SemaphoreType.DMA,
            pltpu.SemaphoreType.DMA,
            pltpu.SemaphoreType.DMA,
            pltpu.SemaphoreType.DMA,
        ],
    )
    def sc_layer(z_hbm, gt_hbm, b_hbm, out_hbm, gt_v, idx_all, acc_v, o_v,
                 b_v, sem_g0, sem_g1, sem_o0, sem_o1):
        sem_g = (sem_g0, sem_g1)
        sem_o = (sem_o0, sem_o1)
        wid = lax.axis_index("s") * NC + lax.axis_index("c")
        base = wid * RPW
        pltpu.sync_copy(b_hbm, b_v)
        pltpu.sync_copy(gt_hbm.at[wid], gt_v)

        def idx_body(g, carry):
            for k in range(DEG):
                for h in range(GRP // 16):
                    idx_all[g, k, pl.ds(h * 16, 16)] = (
                        gt_v[k, pl.ds(g * GRP + h * 16, 16)] + k * stride)
            return carry

        lax.fori_loop(0, NG, idx_body, 0)

        def init_bias(slot):
            def bias_body(rb, carry):
                for rr in range(4):
                    for c in range(UNITS // 16):
                        acc_v[slot, rb * 4 + rr, pl.ds(c * 16, 16)] = (
                            b_v[pl.ds(c * 16, 16)])
                return carry
            lax.fori_loop(0, GRP // 4, bias_body, 0)

        def fire_group(g, slot):
            for k in range(DEG):
                pltpu.async_copy(z_hbm.at[idx_all.at[g, k]], acc_v.at[slot],
                                 sem_g[slot], add=True)

        def wait_group(slot):
            for k in range(DEG):
                pltpu.make_async_copy(
                    z_hbm.at[idx_all.at[0, 0]], acc_v.at[slot],
                    sem_g[slot]).wait()

        def selu(acc):
            return jnp.where(
                acc > 0.0,
                _SELU_SCALE * acc,
                (_SELU_SCALE * _SELU_ALPHA) * (jnp.exp(acc) - 1.0))

        for slot in range(RING):
            init_bias(slot)
        for g in range(RING):
            fire_group(g, g)

        for g in range(NG):
            slot = g % RING
            wait_group(slot)
            if g >= RING:
                pltpu.make_async_copy(
                    o_v.at[slot], out_hbm.at[pl.ds(base, GRP)],
                    sem_o[slot]).wait()

            def ep_body(rb, carry, slot=slot):
                for rr in range(4):
                    for c in range(UNITS // 16):
                        o_v[slot, rb * 4 + rr, pl.ds(c * 16, 16)] = selu(
                            acc_v[slot, rb * 4 + rr, pl.ds(c * 16, 16)])
                return carry

            lax.fori_loop(0, GRP // 4, ep_body, 0)
            if g + RING < NG:
                init_bias(slot)
                fire_group(g + RING, slot)
            pltpu.async_copy(
                o_v.at[slot], out_hbm.at[pl.ds(base + g * GRP, GRP)],
                sem_o[slot])

        for g in range(NG - RING, NG):
            slot = g % RING
            pltpu.make_async_copy(
                o_v.at[slot], out_hbm.at[pl.ds(base, GRP)],
                sem_o[slot]).wait()

    return sc_layer


_sc_layer_1 = _make_sc_layer(NPAD)
_sc_layer_2 = _make_sc_layer(NPAD)


def kernel(X, G, W1, b1, W2, b2, W3, b3):
    Xp = jnp.zeros((NPAD, D), jnp.float32).at[:N].set(X)
    GpT = jnp.zeros((DEG, NPAD), jnp.int32).at[:, :N].set(G.T)
    GpT = GpT.reshape(DEG, NW, RPW).transpose(1, 0, 2)
    W1P = _prep_w(W1, D)
    W2P = _prep_w(W2, UNITS)

    Z1 = _matmul_packed(Xp, W1P)
    H1 = _sc_layer_1(Z1, GpT, b1)
    Z2 = _matmul_packed(H1, W2P)
    H2 = _sc_layer_2(Z2, GpT, b2)
    P = _head(H2, W3, b3.reshape(1, OUT), 1024)
    return P[:N - 1]


# speedup vs baseline: 1.1667x; 1.0057x over previous
import functools

import jax
import jax.numpy as jnp
from jax import lax
from jax.experimental import pallas as pl
from jax.experimental.pallas import tpu as pltpu
from jax.experimental.pallas import tpu_sc as plsc

N = 10000
DEG = 32
D = 128
UNITS = 128
OUT = 3

NC = 2
NS = 16
NW = NC * NS
RPW = 320
NPAD = NW * RPW
BATCH = 4
_SELU_SCALE = 1.0507009873554805
_SELU_ALPHA = 1.6732632423543772

def _prep_w(w, d_in):
    return w.reshape(DEG, d_in, UNITS).transpose(1, 0, 2).reshape(
        d_in, DEG * UNITS)


def _matmul(x, w):
    x = x.astype(jnp.bfloat16)
    w = w.astype(jnp.bfloat16)
    m = x.shape[0]

    def mm(x_ref, w_ref, o_ref):
        o_ref[...] = jnp.dot(x_ref[...], w_ref[...],
                             preferred_element_type=jnp.float32)

    return pl.pallas_call(
        mm,
        grid=(DEG,),
        in_specs=[
            pl.BlockSpec((m, D), lambda kk: (0, 0)),
            pl.BlockSpec((D, UNITS), lambda kk: (0, kk)),
        ],
        out_specs=pl.BlockSpec((m, UNITS), lambda kk: (kk, 0)),
        out_shape=jax.ShapeDtypeStruct((DEG * m, UNITS), jnp.float32),
    )(x, w)


def _head(h, w3, b3, bt):
    m = h.shape[0]

    def hk(h_ref, w_ref, b_ref, o_ref):
        logits = jnp.dot(h_ref[...], w_ref[...],
                         preferred_element_type=jnp.float32) + b_ref[...]
        mx = jnp.max(logits, axis=-1, keepdims=True)
        e = jnp.exp(logits - mx)
        o_ref[...] = e / jnp.sum(e, axis=-1, keepdims=True)

    return pl.pallas_call(
        hk,
        grid=(m // bt,),
        in_specs=[
            pl.BlockSpec((bt, UNITS), lambda i: (i, 0)),
            pl.BlockSpec((UNITS, OUT), lambda i: (0, 0)),
            pl.BlockSpec((1, OUT), lambda i: (0, 0)),
        ],
        out_specs=pl.BlockSpec((bt, OUT), lambda i: (i, 0)),
        out_shape=jax.ShapeDtypeStruct((m, OUT), jnp.float32),
    )(h, w3, b3)


RING = 2
GRP = 32
NG = RPW // GRP


def _make_sc_layer(stride):
    mesh = plsc.VectorSubcoreMesh(core_axis_name="c", subcore_axis_name="s")

    @functools.partial(
        pl.kernel,
        mesh=mesh,
        out_type=jax.ShapeDtypeStruct((NPAD, UNITS), jnp.float32),
        scratch_types=[
            pltpu.VMEM((DEG, RPW), jnp.int32),
            pltpu.VMEM((NG, DEG, GRP), jnp.int32),
            pltpu.VMEM((RING, GRP, UNITS), jnp.float32),
            pltpu.VMEM((RING, GRP, UNITS), jnp.float32),
            pltpu.VMEM((UNITS,), jnp.float32),
            pltpu.SemaphoreType.DMA,
            pltpu.SemaphoreType.DMA,
            pltpu.SemaphoreType.DMA,
            pltpu.SemaphoreType.DMA,
        ],
    )
    def sc_layer(z_hbm, gt_hbm, b_hbm, out_hbm, gt_v, idx_all, acc_v, o_v,
                 b_v, sem_g0, sem_g1, sem_o0, sem_o1):
        sem_g = (sem_g0, sem_g1)
        sem_o = (sem_o0, sem_o1)
        wid = lax.axis_index("s") * NC + lax.axis_index("c")
        base = wid * RPW
        pltpu.sync_copy(b_hbm, b_v)
        pltpu.sync_copy(gt_hbm.at[wid], gt_v)

        def idx_body(g, carry):
            for k in range(DEG):
                for h in range(GRP // 16):
                    idx_all[g, k, pl.ds(h * 16, 16)] = (
                        gt_v[k, pl.ds(g * GRP + h * 16, 16)] + k * stride)
            return carry

        lax.fori_loop(0, NG, idx_body, 0)

        def init_bias(slot):
            def bias_body(rb, carry):
                for rr in range(4):
                    for c in range(UNITS // 16):
                        acc_v[slot, rb * 4 + rr, pl.ds(c * 16, 16)] = (
                            b_v[pl.ds(c * 16, 16)])
                return carry
            lax.fori_loop(0, GRP // 4, bias_body, 0)

        def fire_group(g, slot):
            for k in range(DEG):
                pltpu.async_copy(z_hbm.at[idx_all.at[g, k]], acc_v.at[slot],
                                 sem_g[slot], add=True)

        def wait_group(slot):
            for k in range(DEG):
                pltpu.make_async_copy(
                    z_hbm.at[idx_all.at[0, 0]], acc_v.at[slot],
                    sem_g[slot]).wait()

        def selu(acc):
            return jnp.where(
                acc > 0.0,
                _SELU_SCALE * acc,
                (_SELU_SCALE * _SELU_ALPHA) * (jnp.exp(acc) - 1.0))

        for slot in range(RING):
            init_bias(slot)
        for g in range(RING):
            fire_group(g, g)

        for g in range(NG):
            slot = g % RING
            wait_group(slot)
            if g >= RING:
                pltpu.make_async_copy(
                    o_v.at[slot], out_hbm.at[pl.ds(base, GRP)],
                    sem_o[slot]).wait()

            def ep_body(rb, carry, slot=slot):
                for rr in range(4):
                    for c in range(UNITS // 16):
                        o_v[slot, rb * 4 + rr, pl.ds(c * 16, 16)] = selu(
                            acc_v[slot, rb * 4 + rr, pl.ds(c * 16, 16)])
                return carry

            lax.fori_loop(0, GRP // 4, ep_body, 0)
            if g + RING < NG:
                init_bias(slot)
                fire_group(g + RING, slot)
            pltpu.async_copy(
                o_v.at[slot], out_hbm.at[pl.ds(base + g * GRP, GRP)],
                sem_o[slot])

        for g in range(NG - RING, NG):
            slot = g % RING
            pltpu.make_async_copy(
                o_v.at[slot], out_hbm.at[pl.ds(base, GRP)],
                sem_o[slot]).wait()

    return sc_layer


_sc_layer_1 = _make_sc_layer(NPAD)
_sc_layer_2 = _make_sc_layer(NPAD)


def kernel(X, G, W1, b1, W2, b2, W3, b3):
    Xp = jnp.zeros((NPAD, D), jnp.float32).at[:N].set(X)
    GpT = jnp.zeros((DEG, NPAD), jnp.int32).at[:, :N].set(G.T)
    GpT = GpT.reshape(DEG, NW, RPW).transpose(1, 0, 2)
    W1P = _prep_w(W1, D)
    W2P = _prep_w(W2, UNITS)

    Z1 = _matmul(Xp, W1P)
    H1 = _sc_layer_1(Z1, GpT, b1)
    Z2 = _matmul(H1, W2P)
    H2 = _sc_layer_2(Z2, GpT, b2)
    P = _head(H2, W3, b3.reshape(1, OUT), 1024)
    return P[:N - 1]
